# baseline (speedup 1.0000x reference)
import functools

import jax
import jax.numpy as jnp
from jax import lax
from jax.experimental import pallas as pl
from jax.experimental.pallas import tpu as pltpu
from jax.experimental.pallas import tpu_sc as plsc

VOCAB = 1000000
EMB = 64
HID = 100
B = 1024
T = 200
N = B * T

HP = 128
G4 = 4 * HP
PK = 2 * EMB

SEG = 2
TSEG = T // SEG
NSEG = B * TSEG

NC = 2
NS = 16
NW = NC * NS
RW = NSEG // NW
CHUNK = 128
NCH = RW // CHUNK


def _sc_gather_kernel(table_hbm, idx_hbm, out_hbm, idx_v, rows_v, sem):
    wid = lax.axis_index("s") * NC + lax.axis_index("c")
    base = wid * RW
    pltpu.sync_copy(idx_hbm.at[pl.ds(base, RW)], idx_v)

    def body(j, carry):
        pltpu.async_copy(
            table_hbm.at[idx_v.at[pl.ds(j * CHUNK, CHUNK)]], rows_v, sem
        ).wait()
        pltpu.sync_copy(rows_v, out_hbm.at[pl.ds(base + j * CHUNK, CHUNK)])
        return carry

    lax.fori_loop(0, NCH, body, 0)


def _sc_gather(table, idx):
    mesh = plsc.VectorSubcoreMesh(core_axis_name="c", subcore_axis_name="s")
    k = functools.partial(
        pl.kernel,
        mesh=mesh,
        out_type=jax.ShapeDtypeStruct((NSEG, PK), jnp.float32),
        scratch_types=[
            pltpu.VMEM((RW,), jnp.int32),
            pltpu.VMEM((CHUNK, PK), jnp.float32),
            pltpu.SemaphoreType.DMA,
        ],
        compiler_params=pltpu.CompilerParams(use_tc_tiling_on_sc=True),
    )(_sc_gather_kernel)
    return k(table, idx)


CB = 32768
OB = CB // 2
RPK_STEPS = -(-VOCAB // CB)
TBL_ROWS = RPK_STEPS * OB
SH_CB = CB.bit_length() - 1
SH_OB = OB.bit_length() - 1


def _repack_body(in_ref, id_ref, out_ref):
    x = in_ref[...]
    idm = id_ref[...]
    dn = (((0,), (0,)), ((), ()))
    a = jax.lax.dot_general(x[:, :OB], idm, dn,
                            preferred_element_type=jnp.float32)
    b = jax.lax.dot_general(x[:, OB:], idm, dn,
                            preferred_element_type=jnp.float32)
    out_ref[...] = jnp.concatenate([a, b], axis=1)


def _repack(emb_t):
    return pl.pallas_call(
        _repack_body,
        grid=(RPK_STEPS,),
        in_specs=[
            pl.BlockSpec((EMB, CB), lambda i: (0, i)),
            pl.BlockSpec((EMB, EMB), lambda i: (0, 0)),
        ],
        out_specs=pl.BlockSpec((OB, PK), lambda i: (i, 0)),
        out_shape=jax.ShapeDtypeStruct((TBL_ROWS, PK), jnp.float32),
        compiler_params=pltpu.CompilerParams(
            dimension_semantics=("arbitrary",),
        ),
    )(emb_t, jnp.eye(EMB, dtype=jnp.float32))


KT = 10
assert TSEG % KT == 0


def _lstm_body(e_ref, par_ref, wx_ref, wh_ref, b_ref, fcw_ref, fcb_ref,
               h0_ref, c0_ref, out_ref, h1_ref, c1_ref, h_ref, c_ref):
    t = pl.program_id(0)

    @pl.when(t == 0)
    def _init():
        h_ref[...] = h0_ref[...]
        c_ref[...] = c0_ref[...]

    h = h_ref[...]
    c = c_ref[...]
    for k in range(KT):
        ep = e_ref[k]
        p = jnp.swapaxes(par_ref[k], 0, 1)
        et = ep[:, :EMB] + (ep[:, EMB:] - ep[:, :EMB]) * p
        gates = jnp.dot(et, wx_ref[...], preferred_element_type=jnp.float32)
        gates = gates + jnp.dot(h, wh_ref[...],
                                preferred_element_type=jnp.float32)
        gates = gates + b_ref[...]
        i = jax.nn.sigmoid(gates[:, 0 * HP:1 * HP])
        f = jax.nn.sigmoid(gates[:, 1 * HP:2 * HP])
        g = jnp.tanh(gates[:, 2 * HP:3 * HP])
        o = jax.nn.sigmoid(gates[:, 3 * HP:4 * HP])
        c = f * c + i * g
        h = o * jnp.tanh(c)
    c_ref[...] = c
    h_ref[...] = h

    @pl.when(t == TSEG // KT - 1)
    def _fin():
        h1_ref[...] = h
        c1_ref[...] = c
        logit = jnp.sum(h * fcw_ref[...], axis=1, keepdims=True) + fcb_ref[...]
        out_ref[...] = jax.nn.sigmoid(logit)


def _lstm_seg(e, par, wx, wh, bias, fcw, fcb, h0, c0):
    return pl.pallas_call(
        _lstm_body,
        grid=(TSEG // KT,),
        in_specs=[
            pl.BlockSpec((KT, B, PK), lambda t: (t, 0, 0)),
            pl.BlockSpec((KT, 1, B), lambda t: (t, 0, 0)),
            pl.BlockSpec((EMB, G4), lambda t: (0, 0)),
            pl.BlockSpec((HP, G4), lambda t: (0, 0)),
            pl.BlockSpec((1, G4), lambda t: (0, 0)),
            pl.BlockSpec((1, HP), lambda t: (0, 0)),
            pl.BlockSpec((1, 1), lambda t: (0, 0)),
            pl.BlockSpec((B, HP), lambda t: (0, 0)),
            pl.BlockSpec((B, HP), lambda t: (0, 0)),
        ],
        out_specs=[
            pl.BlockSpec((B, 1), lambda t: (0, 0)),
            pl.BlockSpec((B, HP), lambda t: (0, 0)),
            pl.BlockSpec((B, HP), lambda t: (0, 0)),
        ],
        out_shape=[
            jax.ShapeDtypeStruct((B, 1), jnp.float32),
            jax.ShapeDtypeStruct((B, HP), jnp.float32),
            jax.ShapeDtypeStruct((B, HP), jnp.float32),
        ],
        scratch_shapes=[
            pltpu.VMEM((B, HP), jnp.float32),
            pltpu.VMEM((B, HP), jnp.float32),
        ],
        compiler_params=pltpu.CompilerParams(
            dimension_semantics=("arbitrary",),
        ),
    )(e, par, wx, wh, bias, fcw, fcb, h0, c0)


def _prep_weights(W_ih, W_hh, b_ih, b_hh, fc_w, fc_b):
    wx = jnp.pad(W_ih.reshape(4, HID, EMB), ((0, 0), (0, HP - HID), (0, 0)))
    wx = wx.transpose(2, 0, 1).reshape(EMB, G4)
    wh = jnp.pad(W_hh.reshape(4, HID, HID),
                 ((0, 0), (0, HP - HID), (0, HP - HID)))
    wh = wh.transpose(2, 0, 1).reshape(HP, G4)
    bias = jnp.pad((b_ih + b_hh).reshape(4, HID),
                   ((0, 0), (0, HP - HID))).reshape(1, G4)
    fcw = jnp.pad(fc_w, ((0, 0), (0, HP - HID)))
    fcb = fc_b.reshape(1, 1)
    return wx, wh, bias, fcw, fcb


def kernel(x, emb, W_ih, W_hh, b_ih, b_hh, fc_w, fc_b):
    xt = x.astype(jnp.int32).T
    idx = (((xt >> SH_CB) << SH_OB) | (xt & (OB - 1))).reshape(SEG, NSEG)
    par = ((xt >> SH_OB) & 1).astype(jnp.float32).reshape(SEG, TSEG, 1, B)
    table = _repack(emb.T)
    wx, wh, bias, fcw, fcb = _prep_weights(W_ih, W_hh, b_ih, b_hh, fc_w, fc_b)

    h = jnp.zeros((B, HP), jnp.float32)
    c = jnp.zeros((B, HP), jnp.float32)
    out = None
    for s in range(SEG):
        e_s = _sc_gather(table, idx[s]).reshape(TSEG, B, PK)
        out, h, c = _lstm_seg(e_s, par[s], wx, wh, bias, fcw, fcb, h, c)
    return out[:, 0]


# speedup vs baseline: 1.0275x; 1.0275x over previous
import functools

import jax
import jax.numpy as jnp
from jax import lax
from jax.experimental import pallas as pl
from jax.experimental.pallas import tpu as pltpu
from jax.experimental.pallas import tpu_sc as plsc

VOCAB = 1000000
EMB = 64
HID = 100
B = 1024
T = 200
N = B * T

HP = 128
G4 = 4 * HP
PK = 2 * EMB

SEG = 5
TSEG = T // SEG
NSEG = B * TSEG

NC = 2
NS = 16
NW = NC * NS
RW = NSEG // NW
CHUNK = 128
NCH = RW // CHUNK


def _sc_gather_kernel(table_hbm, idx_hbm, out_hbm, idx_v, rows_v, sem):
    wid = lax.axis_index("s") * NC + lax.axis_index("c")
    base = wid * RW
    pltpu.sync_copy(idx_hbm.at[pl.ds(base, RW)], idx_v)

    def body(j, carry):
        pltpu.async_copy(
            table_hbm.at[idx_v.at[pl.ds(j * CHUNK, CHUNK)]], rows_v, sem
        ).wait()
        pltpu.sync_copy(rows_v, out_hbm.at[pl.ds(base + j * CHUNK, CHUNK)])
        return carry

    lax.fori_loop(0, NCH, body, 0)


def _sc_gather(table, idx):
    mesh = plsc.VectorSubcoreMesh(core_axis_name="c", subcore_axis_name="s")
    k = functools.partial(
        pl.kernel,
        mesh=mesh,
        out_type=jax.ShapeDtypeStruct((NSEG, PK), jnp.float32),
        scratch_types=[
            pltpu.VMEM((RW,), jnp.int32),
            pltpu.VMEM((CHUNK, PK), jnp.float32),
            pltpu.SemaphoreType.DMA,
        ],
        compiler_params=pltpu.CompilerParams(use_tc_tiling_on_sc=True),
    )(_sc_gather_kernel)
    return k(table, idx)


CB = 32768
OB = CB // 2
RPK_STEPS = -(-VOCAB // CB)
TBL_ROWS = RPK_STEPS * OB
SH_CB = CB.bit_length() - 1
SH_OB = OB.bit_length() - 1


def _repack_body(in_ref, id_ref, out_ref):
    x = in_ref[...]
    idm = id_ref[...]
    dn = (((0,), (0,)), ((), ()))
    a = jax.lax.dot_general(x[:, :OB], idm, dn,
                            preferred_element_type=jnp.float32)
    b = jax.lax.dot_general(x[:, OB:], idm, dn,
                            preferred_element_type=jnp.float32)
    out_ref[...] = jnp.concatenate([a, b], axis=1)


def _repack(emb_t):
    return pl.pallas_call(
        _repack_body,
        grid=(RPK_STEPS,),
        in_specs=[
            pl.BlockSpec((EMB, CB), lambda i: (0, i)),
            pl.BlockSpec((EMB, EMB), lambda i: (0, 0)),
        ],
        out_specs=pl.BlockSpec((OB, PK), lambda i: (i, 0)),
        out_shape=jax.ShapeDtypeStruct((TBL_ROWS, PK), jnp.float32),
        compiler_params=pltpu.CompilerParams(
            dimension_semantics=("arbitrary",),
        ),
    )(emb_t, jnp.eye(EMB, dtype=jnp.float32))


KT = 10
assert TSEG % KT == 0


def _lstm_body(e_ref, par_ref, wx_ref, wh_ref, b_ref, fcw_ref, fcb_ref,
               h0_ref, c0_ref, out_ref, h1_ref, c1_ref, h_ref, c_ref):
    t = pl.program_id(0)

    @pl.when(t == 0)
    def _init():
        h_ref[...] = h0_ref[...]
        c_ref[...] = c0_ref[...]

    h = h_ref[...]
    c = c_ref[...]
    for k in range(KT):
        ep = e_ref[k]
        p = jnp.swapaxes(par_ref[k], 0, 1)
        et = ep[:, :EMB] + (ep[:, EMB:] - ep[:, :EMB]) * p
        gates = jnp.dot(et, wx_ref[...], preferred_element_type=jnp.float32)
        gates = gates + jnp.dot(h, wh_ref[...],
                                preferred_element_type=jnp.float32)
        gates = gates + b_ref[...]
        i = jax.nn.sigmoid(gates[:, 0 * HP:1 * HP])
        f = jax.nn.sigmoid(gates[:, 1 * HP:2 * HP])
        g = jnp.tanh(gates[:, 2 * HP:3 * HP])
        o = jax.nn.sigmoid(gates[:, 3 * HP:4 * HP])
        c = f * c + i * g
        h = o * jnp.tanh(c)
    c_ref[...] = c
    h_ref[...] = h

    @pl.when(t == TSEG // KT - 1)
    def _fin():
        h1_ref[...] = h
        c1_ref[...] = c
        logit = jnp.sum(h * fcw_ref[...], axis=1, keepdims=True) + fcb_ref[...]
        out_ref[...] = jax.nn.sigmoid(logit)


def _lstm_seg(e, par, wx, wh, bias, fcw, fcb, h0, c0):
    return pl.pallas_call(
        _lstm_body,
        grid=(TSEG // KT,),
        in_specs=[
            pl.BlockSpec((KT, B, PK), lambda t: (t, 0, 0)),
            pl.BlockSpec((KT, 1, B), lambda t: (t, 0, 0)),
            pl.BlockSpec((EMB, G4), lambda t: (0, 0)),
            pl.BlockSpec((HP, G4), lambda t: (0, 0)),
            pl.BlockSpec((1, G4), lambda t: (0, 0)),
            pl.BlockSpec((1, HP), lambda t: (0, 0)),
            pl.BlockSpec((1, 1), lambda t: (0, 0)),
            pl.BlockSpec((B, HP), lambda t: (0, 0)),
            pl.BlockSpec((B, HP), lambda t: (0, 0)),
        ],
        out_specs=[
            pl.BlockSpec((B, 1), lambda t: (0, 0)),
            pl.BlockSpec((B, HP), lambda t: (0, 0)),
            pl.BlockSpec((B, HP), lambda t: (0, 0)),
        ],
        out_shape=[
            jax.ShapeDtypeStruct((B, 1), jnp.float32),
            jax.ShapeDtypeStruct((B, HP), jnp.float32),
            jax.ShapeDtypeStruct((B, HP), jnp.float32),
        ],
        scratch_shapes=[
            pltpu.VMEM((B, HP), jnp.float32),
            pltpu.VMEM((B, HP), jnp.float32),
        ],
        compiler_params=pltpu.CompilerParams(
            dimension_semantics=("arbitrary",),
        ),
    )(e, par, wx, wh, bias, fcw, fcb, h0, c0)


def _prep_weights(W_ih, W_hh, b_ih, b_hh, fc_w, fc_b):
    wx = jnp.pad(W_ih.reshape(4, HID, EMB), ((0, 0), (0, HP - HID), (0, 0)))
    wx = wx.transpose(2, 0, 1).reshape(EMB, G4)
    wh = jnp.pad(W_hh.reshape(4, HID, HID),
                 ((0, 0), (0, HP - HID), (0, HP - HID)))
    wh = wh.transpose(2, 0, 1).reshape(HP, G4)
    bias = jnp.pad((b_ih + b_hh).reshape(4, HID),
                   ((0, 0), (0, HP - HID))).reshape(1, G4)
    fcw = jnp.pad(fc_w, ((0, 0), (0, HP - HID)))
    fcb = fc_b.reshape(1, 1)
    return wx, wh, bias, fcw, fcb


def kernel(x, emb, W_ih, W_hh, b_ih, b_hh, fc_w, fc_b):
    xt = x.astype(jnp.int32).T
    idx = (((xt >> SH_CB) << SH_OB) | (xt & (OB - 1))).reshape(SEG, NSEG)
    par = ((xt >> SH_OB) & 1).astype(jnp.float32).reshape(SEG, TSEG, 1, B)
    table = _repack(emb.T)
    wx, wh, bias, fcw, fcb = _prep_weights(W_ih, W_hh, b_ih, b_hh, fc_w, fc_b)

    h = jnp.zeros((B, HP), jnp.float32)
    c = jnp.zeros((B, HP), jnp.float32)
    out = None
    for s in range(SEG):
        e_s = _sc_gather(table, idx[s]).reshape(TSEG, B, PK)
        out, h, c = _lstm_seg(e_s, par[s], wx, wh, bias, fcw, fcb, h, c)
    return out[:, 0]
